# Initial kernel scaffold; baseline (speedup 1.0000x reference)
import jax
import jax.numpy as jnp
from jax.experimental import pallas as pl


def kernel(y_true, y_pred):
    raise NotImplementedError("write your pallas kernel here")


# speedup vs baseline: 4.5116x; 4.5116x over previous
import functools

import jax
import jax.numpy as jnp
from jax.experimental import pallas as pl
from jax.experimental.pallas import tpu as pltpu


def _body(y_ref, x_ref, o_ref):
    x = x_ref[...]
    y = y_ref[...]
    shape3 = x.shape
    y3 = jax.lax.broadcast_in_dim(y, shape3, (0, 1))
    mask3 = y3 != -1
    safe3 = jnp.where(mask3, y3, 0)
    iota3 = jax.lax.broadcasted_iota(jnp.int32, shape3, 2)
    g = jnp.sum(jnp.where(iota3 == safe3, x, 0.0), axis=-1)
    lse = jnp.log(jnp.sum(jnp.exp(x), axis=-1))
    loss = jnp.where(y != -1, lse - g, 0.0)
    o_ref[0] = jnp.sum(loss, axis=-1, keepdims=True)


def kernel(y_true, y_pred):
    b, a, c = y_pred.shape
    bbb = 64
    nblk = b // bbb

    out = pl.pallas_call(
        _body,
        grid=(nblk,),
        in_specs=[
            pl.BlockSpec((bbb, a), lambda i: (i, 0)),
            pl.BlockSpec((bbb, a, c), lambda i: (i, 0, 0)),
        ],
        out_specs=pl.BlockSpec((1, bbb, 1), lambda i: (i, 0, 0)),
        out_shape=jax.ShapeDtypeStruct((nblk, bbb, 1), jnp.float32),
    )(y_true.astype(jnp.int32), y_pred)
    return out.reshape(b)


# speedup vs baseline: 5.0607x; 1.1217x over previous
import functools

import jax
import jax.numpy as jnp
from jax.experimental import pallas as pl
from jax.experimental.pallas import tpu as pltpu


def _body(y_ref, x_ref, o_ref):
    x = x_ref[...]
    y = y_ref[...]
    shape3 = x.shape
    y3 = jax.lax.broadcast_in_dim(y, shape3, (0, 1))
    mask3 = y3 != -1
    safe3 = jnp.where(mask3, y3, 0)
    iota3 = jax.lax.broadcasted_iota(jnp.int32, shape3, 2)
    g = jnp.sum(jnp.where(iota3 == safe3, x, 0.0), axis=-1)
    lse = jnp.log(jnp.sum(jnp.exp(x), axis=-1))
    loss = jnp.where(y != -1, lse - g, 0.0)
    o_ref[0] = jnp.sum(loss, axis=-1, keepdims=True)


def kernel(y_true, y_pred):
    b, a, c = y_pred.shape
    bbb = 128
    nblk = b // bbb

    out = pl.pallas_call(
        _body,
        grid=(nblk,),
        in_specs=[
            pl.BlockSpec((bbb, a), lambda i: (i, 0)),
            pl.BlockSpec((bbb, a, c), lambda i: (i, 0, 0)),
        ],
        out_specs=pl.BlockSpec((1, bbb, 1), lambda i: (i, 0, 0)),
        out_shape=jax.ShapeDtypeStruct((nblk, bbb, 1), jnp.float32),
    )(y_true.astype(jnp.int32), y_pred)
    return out.reshape(b)


# speedup vs baseline: 5.0637x; 1.0006x over previous
import functools

import jax
import jax.numpy as jnp
from jax.experimental import pallas as pl
from jax.experimental.pallas import tpu as pltpu


def _body(y_ref, x_ref, o_ref):
    x = x_ref[...]
    y = y_ref[...]
    shape3 = x.shape
    y3 = jax.lax.broadcast_in_dim(y, shape3, (0, 1))
    mask3 = y3 != -1
    safe3 = jnp.where(mask3, y3, 0)
    iota3 = jax.lax.broadcasted_iota(jnp.int32, shape3, 2)
    g = jnp.sum(jnp.where(iota3 == safe3, x, 0.0), axis=-1)
    lse = jnp.log(jnp.sum(jnp.exp(x), axis=-1))
    loss = jnp.where(y != -1, lse - g, 0.0)
    o_ref[0] = jnp.sum(loss, axis=-1, keepdims=True)


def kernel(y_true, y_pred):
    b, a, c = y_pred.shape
    bbb = 256
    nblk = b // bbb

    out = pl.pallas_call(
        _body,
        grid=(nblk,),
        in_specs=[
            pl.BlockSpec((bbb, a), lambda i: (i, 0)),
            pl.BlockSpec((bbb, a, c), lambda i: (i, 0, 0)),
        ],
        out_specs=pl.BlockSpec((1, bbb, 1), lambda i: (i, 0, 0)),
        out_shape=jax.ShapeDtypeStruct((nblk, bbb, 1), jnp.float32),
    )(y_true.astype(jnp.int32), y_pred)
    return out.reshape(b)
